# baseline (speedup 1.0000x reference)
import functools

import jax
import jax.numpy as jnp
from jax import lax
from jax.experimental import pallas as pl
from jax.experimental.pallas import tpu as pltpu
from jax.experimental.pallas import tpu_sc as plsc

_NW = 32
_CH = 8192
_UNROLL = 8


def _sc_scatter(idx_t, src_t, m_rows):
    d_cols, n_upd = idx_t.shape
    half = m_rows // 2
    cols_per_worker = d_cols // _NW
    n_chunks = n_upd // _CH

    mesh = plsc.VectorSubcoreMesh(core_axis_name="c", subcore_axis_name="s")

    @functools.partial(
        pl.kernel,
        out_type=jax.ShapeDtypeStruct((d_cols, m_rows), jnp.float32),
        mesh=mesh,
        scratch_types=[
            pltpu.VMEM((half,), jnp.float32),
            pltpu.VMEM((_CH,), jnp.int32),
            pltpu.VMEM((_CH,), jnp.float32),
            pltpu.VMEM((_CH,), jnp.int32),
            pltpu.VMEM((_CH,), jnp.float32),
            pltpu.SemaphoreType.DMA,
            pltpu.SemaphoreType.DMA,
        ],
        compiler_params=pltpu.CompilerParams(needs_layout_passes=False),
    )
    def scatter_kernel(idx_hbm, src_hbm, delta_hbm,
                       acc, ibuf0, sbuf0, ibuf1, sbuf1, sem0, sem1):
        wid = lax.axis_index("s") * 2 + lax.axis_index("c")
        zeros16 = jnp.zeros((16,), jnp.float32)
        bufs = ((ibuf0, sbuf0, sem0), (ibuf1, sbuf1, sem1))

        def run_half(is_high):
            lo = half if is_high else 0

            def task_body(t, carry):
                col = t * _NW + wid

                def fire(ch, b):
                    ib, sb, sem = bufs[b]
                    pltpu.async_copy(
                        idx_hbm.at[col, pl.ds(ch * _CH, _CH)], ib, sem)
                    pltpu.async_copy(
                        src_hbm.at[col, pl.ds(ch * _CH, _CH)], sb, sem)

                def drain(ch, b):
                    ib, sb, sem = bufs[b]
                    pltpu.make_async_copy(
                        idx_hbm.at[col, pl.ds(ch * _CH, _CH)], ib, sem).wait()
                    pltpu.make_async_copy(
                        src_hbm.at[col, pl.ds(ch * _CH, _CH)], sb, sem).wait()

                fire(0, 0)

                @plsc.parallel_loop(0, half // 16, 1, unroll=_UNROLL)
                def zero_body(i):
                    acc[pl.ds(i * 16, 16)] = zeros16

                for ch in range(n_chunks):
                    b = ch % 2
                    if ch + 1 < n_chunks:
                        fire(ch + 1, 1 - b)
                    drain(ch, b)
                    ib, sb, _ = bufs[b]

                    @plsc.parallel_loop(0, _CH // 16, 1, unroll=_UNROLL)
                    def vec_body(k, ib=ib, sb=sb):
                        iv = ib[pl.ds(k * 16, 16)]
                        sv = sb[pl.ds(k * 16, 16)]
                        if is_high:
                            local = iv - half
                            msk = iv >= half
                        else:
                            local = iv
                            msk = iv < half
                        plsc.addupdate_scatter(acc, [local], sv, mask=msk)

                pltpu.sync_copy(acc, delta_hbm.at[col, pl.ds(lo, half)])
                return carry

            lax.fori_loop(0, cols_per_worker, task_body, 0)

        run_half(False)
        run_half(True)

    return scatter_kernel(idx_t, src_t)


def _combine2(inp, d1, d2):
    m_rows, d_cols = inp.shape
    dg = d_cols // 2
    bm = 4096

    def body(inp_ref, d1_ref, d2_ref, out_ref):
        out_ref[:, :dg] = inp_ref[:, :dg] + d1_ref[...].T
        out_ref[:, dg:] = inp_ref[:, dg:] + d2_ref[...].T

    return pl.pallas_call(
        body,
        grid=(m_rows // bm,),
        in_specs=[
            pl.BlockSpec((bm, d_cols), lambda i: (i, 0)),
            pl.BlockSpec((dg, bm), lambda i: (0, i)),
            pl.BlockSpec((dg, bm), lambda i: (0, i)),
        ],
        out_specs=pl.BlockSpec((bm, d_cols), lambda i: (i, 0)),
        out_shape=jax.ShapeDtypeStruct((m_rows, d_cols), jnp.float32),
    )(inp, d1, d2)


def kernel(inp, idx, src):
    m_rows, d_cols = inp.shape
    dg = d_cols // 2
    idx = idx.astype(jnp.int32)
    d1 = _sc_scatter(idx[:, :dg].T, src[:, :dg].T, m_rows)
    d2 = _sc_scatter(idx[:, dg:].T, src[:, dg:].T, m_rows)
    return _combine2(inp, d1, d2)


# speedup vs baseline: 1.1317x; 1.1317x over previous
import functools

import jax
import jax.numpy as jnp
from jax import lax
from jax.experimental import pallas as pl
from jax.experimental.pallas import tpu as pltpu
from jax.experimental.pallas import tpu_sc as plsc

_NW = 32
_CH = 8192
_UNROLL = 8


def _sc_scatter(idx_t, src_t, m_rows):
    d_cols, n_upd = idx_t.shape
    half = m_rows // 2
    cols_per_worker = d_cols // _NW
    n_chunks = n_upd // _CH

    mesh = plsc.VectorSubcoreMesh(core_axis_name="c", subcore_axis_name="s")

    @functools.partial(
        pl.kernel,
        out_type=jax.ShapeDtypeStruct((d_cols, m_rows), jnp.float32),
        mesh=mesh,
        scratch_types=[
            pltpu.VMEM((half,), jnp.float32),
            pltpu.VMEM((_CH,), jnp.int32),
            pltpu.VMEM((_CH,), jnp.float32),
            pltpu.VMEM((_CH,), jnp.int32),
            pltpu.VMEM((_CH,), jnp.float32),
            pltpu.SemaphoreType.DMA,
            pltpu.SemaphoreType.DMA,
        ],
        compiler_params=pltpu.CompilerParams(needs_layout_passes=False),
    )
    def scatter_kernel(idx_hbm, src_hbm, delta_hbm,
                       acc, ibuf0, sbuf0, ibuf1, sbuf1, sem0, sem1):
        wid = lax.axis_index("s") * 2 + lax.axis_index("c")
        zeros16 = jnp.zeros((16,), jnp.float32)
        bufs = ((ibuf0, sbuf0, sem0), (ibuf1, sbuf1, sem1))

        def run_half(is_high):
            lo = half if is_high else 0

            def task_body(t, carry):
                col = t * _NW + wid

                def fire(ch, b):
                    ib, sb, sem = bufs[b]
                    pltpu.async_copy(
                        idx_hbm.at[col, pl.ds(ch * _CH, _CH)], ib, sem)
                    pltpu.async_copy(
                        src_hbm.at[col, pl.ds(ch * _CH, _CH)], sb, sem)

                def drain(ch, b):
                    ib, sb, sem = bufs[b]
                    pltpu.make_async_copy(
                        idx_hbm.at[col, pl.ds(ch * _CH, _CH)], ib, sem).wait()
                    pltpu.make_async_copy(
                        src_hbm.at[col, pl.ds(ch * _CH, _CH)], sb, sem).wait()

                fire(0, 0)

                @plsc.parallel_loop(0, half // 16, 1, unroll=_UNROLL)
                def zero_body(i):
                    acc[pl.ds(i * 16, 16)] = zeros16

                for ch in range(n_chunks):
                    b = ch % 2
                    if ch + 1 < n_chunks:
                        fire(ch + 1, 1 - b)
                    drain(ch, b)
                    ib, sb, _ = bufs[b]

                    @plsc.parallel_loop(0, _CH // 16, 1, unroll=_UNROLL)
                    def vec_body(k, ib=ib, sb=sb):
                        iv = ib[pl.ds(k * 16, 16)]
                        sv = sb[pl.ds(k * 16, 16)]
                        if is_high:
                            local = iv - half
                            msk = iv >= half
                        else:
                            local = iv
                            msk = iv < half
                        plsc.addupdate_scatter(acc, [local], sv, mask=msk)

                pltpu.sync_copy(acc, delta_hbm.at[col, pl.ds(lo, half)])
                return carry

            lax.fori_loop(0, cols_per_worker, task_body, 0)

        run_half(False)
        run_half(True)

    return scatter_kernel(idx_t, src_t)


def _combine(inp, delta_t):
    m_rows, d_cols = inp.shape
    bm = 4096

    def body(inp_ref, dt_ref, out_ref):
        out_ref[...] = inp_ref[...] + dt_ref[...].T

    return pl.pallas_call(
        body,
        grid=(m_rows // bm,),
        in_specs=[
            pl.BlockSpec((bm, d_cols), lambda i: (i, 0)),
            pl.BlockSpec((d_cols, bm), lambda i: (0, i)),
        ],
        out_specs=pl.BlockSpec((bm, d_cols), lambda i: (i, 0)),
        out_shape=jax.ShapeDtypeStruct((m_rows, d_cols), jnp.float32),
    )(inp, delta_t)


def kernel(inp, idx, src):
    m_rows, _ = inp.shape
    idx_t = idx.astype(jnp.int32).T
    src_t = src.T
    delta_t = _sc_scatter(idx_t, src_t, m_rows)
    return _combine(inp, delta_t)


# speedup vs baseline: 1.1804x; 1.0430x over previous
import functools

import jax
import jax.numpy as jnp
from jax import lax
from jax.experimental import pallas as pl
from jax.experimental.pallas import tpu as pltpu
from jax.experimental.pallas import tpu_sc as plsc

_NW = 32
_CH = 8192
_UNROLL = 8


def _sc_scatter(idx_t, src_t, m_rows):
    d_cols, n_upd = idx_t.shape
    half = m_rows // 2
    cols_per_worker = d_cols // _NW
    n_chunks = n_upd // _CH

    mesh = plsc.VectorSubcoreMesh(core_axis_name="c", subcore_axis_name="s")

    @functools.partial(
        pl.kernel,
        out_type=jax.ShapeDtypeStruct((d_cols, m_rows), jnp.float32),
        mesh=mesh,
        scratch_types=[
            pltpu.VMEM((half,), jnp.float32),
            pltpu.VMEM((_CH,), jnp.int32),
            pltpu.VMEM((_CH,), jnp.float32),
            pltpu.VMEM((_CH,), jnp.int32),
            pltpu.VMEM((_CH,), jnp.float32),
            pltpu.SemaphoreType.DMA,
            pltpu.SemaphoreType.DMA,
        ],
        compiler_params=pltpu.CompilerParams(needs_layout_passes=False),
    )
    def scatter_kernel(idx_hbm, src_hbm, delta_hbm,
                       acc, ibuf0, sbuf0, ibuf1, sbuf1, sem0, sem1):
        wid = lax.axis_index("s") * 2 + lax.axis_index("c")
        zeros16 = jnp.zeros((16,), jnp.float32)
        bufs = ((ibuf0, sbuf0, sem0), (ibuf1, sbuf1, sem1))

        def run_half(is_high):
            lo = half if is_high else 0

            def task_body(t, carry):
                col = t * _NW + wid

                def fire(ch, b):
                    ib, sb, sem = bufs[b]
                    pltpu.async_copy(
                        idx_hbm.at[col, pl.ds(ch * _CH, _CH)], ib, sem)
                    pltpu.async_copy(
                        src_hbm.at[col, pl.ds(ch * _CH, _CH)], sb, sem)

                def drain(ch, b):
                    ib, sb, sem = bufs[b]
                    pltpu.make_async_copy(
                        idx_hbm.at[col, pl.ds(ch * _CH, _CH)], ib, sem).wait()
                    pltpu.make_async_copy(
                        src_hbm.at[col, pl.ds(ch * _CH, _CH)], sb, sem).wait()

                fire(0, 0)

                @plsc.parallel_loop(0, half // 16, 1, unroll=_UNROLL)
                def zero_body(i):
                    acc[pl.ds(i * 16, 16)] = zeros16

                for ch in range(n_chunks):
                    b = ch % 2
                    if ch + 1 < n_chunks:
                        fire(ch + 1, 1 - b)
                    drain(ch, b)
                    ib, sb, _ = bufs[b]

                    @plsc.parallel_loop(0, _CH // 16, 1, unroll=_UNROLL)
                    def vec_body(k, ib=ib, sb=sb):
                        iv = ib[pl.ds(k * 16, 16)]
                        sv = sb[pl.ds(k * 16, 16)]
                        if is_high:
                            local = iv - half
                            msk = iv >= half
                        else:
                            local = iv
                            msk = iv < half
                        plsc.addupdate_scatter(acc, [local], sv, mask=msk)

                pltpu.sync_copy(acc, delta_hbm.at[col, pl.ds(lo, half)])
                return carry

            lax.fori_loop(0, cols_per_worker, task_body, 0)

        run_half(False)
        run_half(True)

    return scatter_kernel(idx_t, src_t)


def _transpose2(idx, src):
    n, d = idx.shape
    bn = 4096

    def body(i_ref, s_ref, it_ref, st_ref):
        it_ref[...] = i_ref[...].T
        st_ref[...] = s_ref[...].T

    return pl.pallas_call(
        body,
        grid=(n // bn,),
        in_specs=[
            pl.BlockSpec((bn, d), lambda i: (i, 0)),
            pl.BlockSpec((bn, d), lambda i: (i, 0)),
        ],
        out_specs=[
            pl.BlockSpec((d, bn), lambda i: (0, i)),
            pl.BlockSpec((d, bn), lambda i: (0, i)),
        ],
        out_shape=[
            jax.ShapeDtypeStruct((d, n), jnp.int32),
            jax.ShapeDtypeStruct((d, n), jnp.float32),
        ],
    )(idx, src)


def _combine(inp, delta_t):
    m_rows, d_cols = inp.shape
    bm = 4096

    def body(inp_ref, dt_ref, out_ref):
        out_ref[...] = inp_ref[...] + dt_ref[...].T

    return pl.pallas_call(
        body,
        grid=(m_rows // bm,),
        in_specs=[
            pl.BlockSpec((bm, d_cols), lambda i: (i, 0)),
            pl.BlockSpec((d_cols, bm), lambda i: (0, i)),
        ],
        out_specs=pl.BlockSpec((bm, d_cols), lambda i: (i, 0)),
        out_shape=jax.ShapeDtypeStruct((m_rows, d_cols), jnp.float32),
    )(inp, delta_t)


def kernel(inp, idx, src):
    m_rows, _ = inp.shape
    idx_t, src_t = _transpose2(idx.astype(jnp.int32), src)
    delta_t = _sc_scatter(idx_t, src_t, m_rows)
    return _combine(inp, delta_t)


# speedup vs baseline: 1.2057x; 1.0214x over previous
import functools

import jax
import jax.numpy as jnp
from jax import lax
from jax.experimental import pallas as pl
from jax.experimental.pallas import tpu as pltpu
from jax.experimental.pallas import tpu_sc as plsc

_NW = 32
_CH = 8192
_UNROLL = 8


def _sc_scatter(idx_t, src_t, m_rows):
    d_cols, n_upd = idx_t.shape
    half = m_rows // 2
    cols_per_worker = d_cols // _NW
    n_chunks = n_upd // _CH

    mesh = plsc.VectorSubcoreMesh(core_axis_name="c", subcore_axis_name="s")

    @functools.partial(
        pl.kernel,
        out_type=jax.ShapeDtypeStruct((d_cols, m_rows), jnp.float32),
        mesh=mesh,
        scratch_types=[
            pltpu.VMEM((half,), jnp.float32),
            pltpu.VMEM((_CH,), jnp.int32),
            pltpu.VMEM((_CH,), jnp.float32),
            pltpu.VMEM((_CH,), jnp.int32),
            pltpu.VMEM((_CH,), jnp.float32),
            pltpu.SemaphoreType.DMA,
            pltpu.SemaphoreType.DMA,
        ],
        compiler_params=pltpu.CompilerParams(needs_layout_passes=False),
    )
    def scatter_kernel(idx_hbm, src_hbm, delta_hbm,
                       acc, ibuf0, sbuf0, ibuf1, sbuf1, sem0, sem1):
        wid = lax.axis_index("s") * 2 + lax.axis_index("c")
        zeros16 = jnp.zeros((16,), jnp.float32)
        bufs = ((ibuf0, sbuf0, sem0), (ibuf1, sbuf1, sem1))

        def run_half(is_high):
            lo = half if is_high else 0

            def task_body(t, carry):
                col = t * _NW + wid

                def fire(ch, b):
                    ib, sb, sem = bufs[b]
                    pltpu.async_copy(
                        idx_hbm.at[col, pl.ds(ch * _CH, _CH)], ib, sem)
                    pltpu.async_copy(
                        src_hbm.at[col, pl.ds(ch * _CH, _CH)], sb, sem)

                def drain(ch, b):
                    ib, sb, sem = bufs[b]
                    pltpu.make_async_copy(
                        idx_hbm.at[col, pl.ds(ch * _CH, _CH)], ib, sem).wait()
                    pltpu.make_async_copy(
                        src_hbm.at[col, pl.ds(ch * _CH, _CH)], sb, sem).wait()

                fire(0, 0)

                @plsc.parallel_loop(0, half // 16, 1, unroll=_UNROLL)
                def zero_body(i):
                    acc[pl.ds(i * 16, 16)] = zeros16

                for ch in range(n_chunks):
                    b = ch % 2
                    if ch + 1 < n_chunks:
                        fire(ch + 1, 1 - b)
                    drain(ch, b)
                    ib, sb, _ = bufs[b]

                    @plsc.parallel_loop(0, _CH // 16, 1, unroll=_UNROLL)
                    def vec_body(k, ib=ib, sb=sb):
                        iv = ib[pl.ds(k * 16, 16)]
                        sv = sb[pl.ds(k * 16, 16)]
                        if is_high:
                            local = iv - half
                            msk = iv >= half
                        else:
                            local = iv
                            msk = iv < half
                        plsc.addupdate_scatter(acc, [local], sv, mask=msk)

                pltpu.sync_copy(acc, delta_hbm.at[col, pl.ds(lo, half)])
                return carry

            lax.fori_loop(0, cols_per_worker, task_body, 0)

        run_half(False)
        run_half(True)

    return scatter_kernel(idx_t, src_t)


def _transpose2(idx, src, g, dg):
    n, _ = idx.shape
    bn = 4096

    def body(i_ref, s_ref, it_ref, st_ref):
        it_ref[...] = i_ref[...].T
        st_ref[...] = s_ref[...].T

    return pl.pallas_call(
        body,
        grid=(n // bn,),
        in_specs=[
            pl.BlockSpec((bn, dg), lambda i, g=g: (i, g)),
            pl.BlockSpec((bn, dg), lambda i, g=g: (i, g)),
        ],
        out_specs=[
            pl.BlockSpec((dg, bn), lambda i: (0, i)),
            pl.BlockSpec((dg, bn), lambda i: (0, i)),
        ],
        out_shape=[
            jax.ShapeDtypeStruct((dg, n), jnp.int32),
            jax.ShapeDtypeStruct((dg, n), jnp.float32),
        ],
    )(idx, src)


def _combine2(inp, d1, d2):
    m_rows, d_cols = inp.shape
    dg = d_cols // 2
    bm = 4096

    def body(inp_ref, d1_ref, d2_ref, out_ref):
        out_ref[:, :dg] = inp_ref[:, :dg] + d1_ref[...].T
        out_ref[:, dg:] = inp_ref[:, dg:] + d2_ref[...].T

    return pl.pallas_call(
        body,
        grid=(m_rows // bm,),
        in_specs=[
            pl.BlockSpec((bm, d_cols), lambda i: (i, 0)),
            pl.BlockSpec((dg, bm), lambda i: (0, i)),
            pl.BlockSpec((dg, bm), lambda i: (0, i)),
        ],
        out_specs=pl.BlockSpec((bm, d_cols), lambda i: (i, 0)),
        out_shape=jax.ShapeDtypeStruct((m_rows, d_cols), jnp.float32),
    )(inp, d1, d2)


def _combine(inp, delta_t):
    m_rows, d_cols = inp.shape
    bm = 4096

    def body(inp_ref, dt_ref, out_ref):
        out_ref[...] = inp_ref[...] + dt_ref[...].T

    return pl.pallas_call(
        body,
        grid=(m_rows // bm,),
        in_specs=[
            pl.BlockSpec((bm, d_cols), lambda i: (i, 0)),
            pl.BlockSpec((d_cols, bm), lambda i: (0, i)),
        ],
        out_specs=pl.BlockSpec((bm, d_cols), lambda i: (i, 0)),
        out_shape=jax.ShapeDtypeStruct((m_rows, d_cols), jnp.float32),
    )(inp, delta_t)


def kernel(inp, idx, src):
    m_rows, d_cols = inp.shape
    dg = d_cols // 2
    idx = idx.astype(jnp.int32)
    i1t, s1t = _transpose2(idx, src, 0, dg)
    i2t, s2t = _transpose2(idx, src, 1, dg)
    d1 = _sc_scatter(i1t, s1t, m_rows)
    d2 = _sc_scatter(i2t, s2t, m_rows)
    return _combine2(inp, d1, d2)


# speedup vs baseline: 1.2102x; 1.0037x over previous
import functools

import jax
import jax.numpy as jnp
from jax import lax
from jax.experimental import pallas as pl
from jax.experimental.pallas import tpu as pltpu
from jax.experimental.pallas import tpu_sc as plsc

_NW = 32
_CH = 8192
_UNROLL = 8


def _sc_scatter(idx_t, src_t, m_rows):
    d_cols, n_upd = idx_t.shape
    half = m_rows // 2
    cols_per_worker = d_cols // _NW
    n_chunks = n_upd // _CH

    mesh = plsc.VectorSubcoreMesh(core_axis_name="c", subcore_axis_name="s")

    @functools.partial(
        pl.kernel,
        out_type=jax.ShapeDtypeStruct((d_cols, m_rows), jnp.float32),
        mesh=mesh,
        scratch_types=[
            pltpu.VMEM((half,), jnp.float32),
            pltpu.VMEM((_CH,), jnp.int32),
            pltpu.VMEM((_CH,), jnp.float32),
            pltpu.VMEM((_CH,), jnp.int32),
            pltpu.VMEM((_CH,), jnp.float32),
            pltpu.SemaphoreType.DMA,
            pltpu.SemaphoreType.DMA,
        ],
        compiler_params=pltpu.CompilerParams(needs_layout_passes=False),
    )
    def scatter_kernel(idx_hbm, src_hbm, delta_hbm,
                       acc, ibuf0, sbuf0, ibuf1, sbuf1, sem0, sem1):
        wid = lax.axis_index("s") * 2 + lax.axis_index("c")
        zeros16 = jnp.zeros((16,), jnp.float32)
        bufs = ((ibuf0, sbuf0, sem0), (ibuf1, sbuf1, sem1))

        def run_half(is_high):
            lo = half if is_high else 0

            def task_body(t, carry):
                col = t * _NW + wid

                def fire(ch, b):
                    ib, sb, sem = bufs[b]
                    pltpu.async_copy(
                        idx_hbm.at[col, pl.ds(ch * _CH, _CH)], ib, sem)
                    pltpu.async_copy(
                        src_hbm.at[col, pl.ds(ch * _CH, _CH)], sb, sem)

                def drain(ch, b):
                    ib, sb, sem = bufs[b]
                    pltpu.make_async_copy(
                        idx_hbm.at[col, pl.ds(ch * _CH, _CH)], ib, sem).wait()
                    pltpu.make_async_copy(
                        src_hbm.at[col, pl.ds(ch * _CH, _CH)], sb, sem).wait()

                fire(0, 0)

                @plsc.parallel_loop(0, half // 16, 1, unroll=_UNROLL)
                def zero_body(i):
                    acc[pl.ds(i * 16, 16)] = zeros16

                for ch in range(n_chunks):
                    b = ch % 2
                    if ch + 1 < n_chunks:
                        fire(ch + 1, 1 - b)
                    drain(ch, b)
                    ib, sb, _ = bufs[b]

                    @plsc.parallel_loop(0, _CH // 16, 1, unroll=_UNROLL)
                    def vec_body(k, ib=ib, sb=sb):
                        iv = ib[pl.ds(k * 16, 16)]
                        sv = sb[pl.ds(k * 16, 16)]
                        if is_high:
                            local = iv - half
                            msk = iv >= half
                        else:
                            local = iv
                            msk = iv < half
                        plsc.addupdate_scatter(acc, [local], sv, mask=msk)

                pltpu.sync_copy(acc, delta_hbm.at[col, pl.ds(lo, half)])
                return carry

            lax.fori_loop(0, cols_per_worker, task_body, 0)

        run_half(False)
        run_half(True)

    return scatter_kernel(idx_t, src_t)


def _transpose2(idx, src, g, dg):
    n, _ = idx.shape
    bn = 4096

    def body(i_ref, s_ref, it_ref, st_ref):
        it_ref[...] = i_ref[...].T
        st_ref[...] = s_ref[...].T

    return pl.pallas_call(
        body,
        grid=(n // bn,),
        in_specs=[
            pl.BlockSpec((bn, dg), lambda i, g=g: (i, g)),
            pl.BlockSpec((bn, dg), lambda i, g=g: (i, g)),
        ],
        out_specs=[
            pl.BlockSpec((dg, bn), lambda i: (0, i)),
            pl.BlockSpec((dg, bn), lambda i: (0, i)),
        ],
        out_shape=[
            jax.ShapeDtypeStruct((dg, n), jnp.int32),
            jax.ShapeDtypeStruct((dg, n), jnp.float32),
        ],
    )(idx, src)


def _combine_init(inp, d1):
    m_rows, d_cols = inp.shape
    dg = d_cols // 2
    bm = 4096

    def body(inp_ref, d1_ref, out_ref):
        out_ref[:, :dg] = inp_ref[:, :dg] + d1_ref[...].T
        out_ref[:, dg:] = inp_ref[:, dg:]

    return pl.pallas_call(
        body,
        grid=(m_rows // bm,),
        in_specs=[
            pl.BlockSpec((bm, d_cols), lambda i: (i, 0)),
            pl.BlockSpec((dg, bm), lambda i: (0, i)),
        ],
        out_specs=pl.BlockSpec((bm, d_cols), lambda i: (i, 0)),
        out_shape=jax.ShapeDtypeStruct((m_rows, d_cols), jnp.float32),
    )(inp, d1)


def _combine_update(out1, d2):
    m_rows, d_cols = out1.shape
    dg = d_cols // 2
    bm = 4096

    def body(o_ref, d2_ref, out_ref):
        out_ref[...] = o_ref[...] + d2_ref[...].T

    return pl.pallas_call(
        body,
        grid=(m_rows // bm,),
        in_specs=[
            pl.BlockSpec((bm, dg), lambda i: (i, 1)),
            pl.BlockSpec((dg, bm), lambda i: (0, i)),
        ],
        out_specs=pl.BlockSpec((bm, dg), lambda i: (i, 1)),
        out_shape=jax.ShapeDtypeStruct((m_rows, d_cols), jnp.float32),
        input_output_aliases={0: 0},
    )(out1, d2)


def _combine(inp, delta_t):
    m_rows, d_cols = inp.shape
    bm = 4096

    def body(inp_ref, dt_ref, out_ref):
        out_ref[...] = inp_ref[...] + dt_ref[...].T

    return pl.pallas_call(
        body,
        grid=(m_rows // bm,),
        in_specs=[
            pl.BlockSpec((bm, d_cols), lambda i: (i, 0)),
            pl.BlockSpec((d_cols, bm), lambda i: (0, i)),
        ],
        out_specs=pl.BlockSpec((bm, d_cols), lambda i: (i, 0)),
        out_shape=jax.ShapeDtypeStruct((m_rows, d_cols), jnp.float32),
    )(inp, delta_t)


def kernel(inp, idx, src):
    m_rows, d_cols = inp.shape
    dg = d_cols // 2
    idx = idx.astype(jnp.int32)
    i1t, s1t = _transpose2(idx, src, 0, dg)
    i2t, s2t = _transpose2(idx, src, 1, dg)
    d1 = _sc_scatter(i1t, s1t, m_rows)
    d2 = _sc_scatter(i2t, s2t, m_rows)
    out1 = _combine_init(inp, d1)
    return _combine_update(out1, d2)
